# baseline (speedup 1.0000x reference)
import jax
import jax.numpy as jnp
from jax import lax
from jax.experimental import pallas as pl
from jax.experimental.pallas import tpu as pltpu
from jax.experimental.pallas import tpu_sc as plsc

N_NODE = 50000
C = 128
H = 2
EPS = 1e-5

_BM = 1000

_NC = 2
_NS = 16
_L = 16
_NW = _NC * _NS
_E = 524288
_EPT = _E // _NW
_ECH = 4096
_NT = 50048


def _ex_body(asrc_hbm, adst_hbm, src_hbm, dst_hbm, b_hbm, ex_hbm,
             tsrc, tdst, sbuf, dbuf, exbuf, bbuf):
    cid = lax.axis_index("c")
    sid = lax.axis_index("s")
    wid = sid * _NC + cid
    base = wid * _EPT
    pltpu.sync_copy(b_hbm, bbuf)
    for h in range(H):
        pltpu.sync_copy(asrc_hbm.at[h], tsrc)
        pltpu.sync_copy(adst_hbm.at[h], tdst)
        bvec = bbuf[h]
        for ch in range(_EPT // _ECH):
            off = base + ch * _ECH
            pltpu.sync_copy(src_hbm.at[pl.ds(off, _ECH)], sbuf)
            pltpu.sync_copy(dst_hbm.at[pl.ds(off, _ECH)], dbuf)

            def body(i, _):
                s = sbuf[pl.ds(i * _L, _L)]
                d = dbuf[pl.ds(i * _L, _L)]
                al = plsc.load_gather(tsrc, [s]) + plsc.load_gather(tdst, [d])
                al = jnp.where(al > 0, al, 0.2 * al) - bvec
                exbuf[pl.ds(i * _L, _L)] = jnp.exp(al)
                return 0

            lax.fori_loop(0, _ECH // _L, body, 0)
            pltpu.sync_copy(exbuf, ex_hbm.at[h, pl.ds(off, _ECH)])


def _edge_ex(a_src_t, a_dst_t, src, dst, bmat):
    return pl.kernel(
        _ex_body,
        out_type=jax.ShapeDtypeStruct((H, _E), jnp.float32),
        mesh=plsc.VectorSubcoreMesh(core_axis_name="c", subcore_axis_name="s"),
        compiler_params=pltpu.CompilerParams(needs_layout_passes=False),
        scratch_types=[
            pltpu.VMEM((_NT,), jnp.float32),
            pltpu.VMEM((_NT,), jnp.float32),
            pltpu.VMEM((_ECH,), jnp.int32),
            pltpu.VMEM((_ECH,), jnp.int32),
            pltpu.VMEM((_ECH,), jnp.float32),
            pltpu.VMEM((H, _L), jnp.float32),
        ],
    )(a_src_t, a_dst_t, src, dst, bmat)


def _proj_body(x_ref, w_ref, att_src_ref, att_dst_ref, h_ref, a_ref):
    h = jnp.dot(x_ref[...], w_ref[...], preferred_element_type=jnp.float32)
    h_ref[...] = h
    hh = h.reshape(-1, H, C)
    a = (hh * att_dst_ref[...]).sum(-1)
    b = (hh * att_src_ref[...]).sum(-1)
    a_ref[...] = jnp.concatenate([b, a], axis=-1)


def _project(x, W, att_src, att_dst):
    n = x.shape[0]
    h, a = pl.pallas_call(
        _proj_body,
        grid=(n // _BM,),
        in_specs=[
            pl.BlockSpec((_BM, C), lambda i: (i, 0)),
            pl.BlockSpec((C, H * C), lambda i: (0, 0)),
            pl.BlockSpec((1, H, C), lambda i: (0, 0, 0)),
            pl.BlockSpec((1, H, C), lambda i: (0, 0, 0)),
        ],
        out_specs=[
            pl.BlockSpec((_BM, H * C), lambda i: (i, 0)),
            pl.BlockSpec((_BM, 2 * H), lambda i: (i, 0)),
        ],
        out_shape=[
            jax.ShapeDtypeStruct((n, H * C), jnp.float32),
            jax.ShapeDtypeStruct((n, 2 * H), jnp.float32),
        ],
    )(x, W, att_src, att_dst)
    return h, a


_R = 184
_CHC = _NS * _R
_NPASS = 9
_NPAD = 2 * _NPASS * _CHC
_SCH = 2048
_SS = (_E // _NS) // _SCH
_CCAP = _SCH + _L
_QCAP = _SCH + _L
_KB = 64
_D = H * C


def _msg_body(hsrc_hbm, src_hbm, dst_hbm, ex_hbm, num_hbm, den_hbm,
              dbuf, sbuf, e0buf, e1buf, c_src, c_ldst, c_e0, c_e1,
              q_src, q_ldst, q_e0, q_e1, g, cbuf, cntv, acc, accd, qref,
              st_src, st_ldst, st_e0, st_e1, st_cnt, sem):
    cid = lax.axis_index("c")
    sid = lax.axis_index("s")
    lane = lax.iota(jnp.int32, _L)
    zi = jnp.zeros((_L,), jnp.int32)
    zf = jnp.zeros((_L,), jnp.float32)
    qfill = jnp.full((_L,), _R, jnp.int32)

    def prefill_q(i, _):
        q_src[pl.ds(i * _L, _L)] = zi
        q_ldst[pl.ds(i * _L, _L)] = qfill
        return 0

    def flush(qoff):
        def batch(b, _):
            @pl.when(b * _KB < qoff)
            def _():
                pltpu.async_copy(
                    hsrc_hbm.at[q_src.at[pl.ds(b * _KB, _KB)]], g, sem
                ).wait()

                def accum(j, _):
                    jj = b * _KB + j
                    gidx = jnp.full((_L,), jj, jnp.int32)
                    e0 = plsc.load_gather(q_e0, [gidx])
                    e1 = plsc.load_gather(q_e1, [gidx])
                    r = plsc.load_gather(q_ldst, [gidx])[0]
                    for q in range(8):
                        plsc.addupdate(acc.at[r, pl.ds(q * _L, _L)],
                                       g[j, pl.ds(q * _L, _L)] * e0)
                    for q in range(8, 16):
                        plsc.addupdate(acc.at[r, pl.ds(q * _L, _L)],
                                       g[j, pl.ds(q * _L, _L)] * e1)
                    dv = (jnp.where(lane == 0, e0, 0.0)
                          + jnp.where(lane == 1, e1, 0.0))
                    plsc.addupdate(accd.at[r], dv)
                    return 0

                lax.fori_loop(0, _KB, accum, 0)
            return 0

        lax.fori_loop(0, _QCAP // _KB + 1, batch, 0)
        lax.fori_loop(0, _QCAP // _L, prefill_q, 0)
        return jnp.int32(0)

    def do_pass(p, _p):
        klo = (2 * p + cid) * _CHC
        olo = sid * _R

        def zacc(r, _):
            for q in range(_D // _L):
                acc[r, pl.ds(q * _L, _L)] = zf
            accd[r, :] = zf
            return 0

        lax.fori_loop(0, _R + 8, zacc, 0)
        lax.fori_loop(0, _QCAP // _L, prefill_q, 0)

        def do_ss(ss, _s):
            ebase = pl.multiple_of(sid * (_E // _NS) + ss * _SCH, 8)
            pltpu.sync_copy(dst_hbm.at[pl.ds(ebase, _SCH)], dbuf)
            pltpu.sync_copy(src_hbm.at[pl.ds(ebase, _SCH)], sbuf)
            pltpu.sync_copy(ex_hbm.at[0, pl.ds(ebase, _SCH)], e0buf)
            pltpu.sync_copy(ex_hbm.at[1, pl.ds(ebase, _SCH)], e1buf)

            def prefill_c(i, _):
                c_src[pl.ds(i * _L, _L)] = zi
                c_ldst[pl.ds(i * _L, _L)] = jnp.full((_L,), _CHC, jnp.int32)
                return 0

            lax.fori_loop(0, _CCAP // _L, prefill_c, 0)

            def scan(i, off):
                d = dbuf[pl.ds(i * _L, _L)]
                m = (d >= klo) & (d < klo + _CHC)
                plsc.store_compressed(c_src.at[pl.ds(off, _L)],
                                      sbuf[pl.ds(i * _L, _L)], mask=m)
                plsc.store_compressed(c_ldst.at[pl.ds(off, _L)],
                                      d - klo, mask=m)
                plsc.store_compressed(c_e0.at[pl.ds(off, _L)],
                                      e0buf[pl.ds(i * _L, _L)], mask=m)
                plsc.store_compressed(c_e1.at[pl.ds(off, _L)],
                                      e1buf[pl.ds(i * _L, _L)], mask=m)
                return off + jnp.sum(m.astype(jnp.int32))

            cnt = lax.fori_loop(0, _SCH // _L, scan, jnp.int32(0))
            so = pl.multiple_of(sid * _CCAP, 8)
            pltpu.sync_copy(c_src, st_src.at[pl.ds(so, _CCAP)])
            pltpu.sync_copy(c_ldst, st_ldst.at[pl.ds(so, _CCAP)])
            pltpu.sync_copy(c_e0, st_e0.at[pl.ds(so, _CCAP)])
            pltpu.sync_copy(c_e1, st_e1.at[pl.ds(so, _CCAP)])
            cbuf[...] = jnp.full((_L,), cnt, jnp.int32)
            pltpu.sync_copy(cbuf, st_cnt.at[pl.ds(sid * _L, _L)])
            plsc.subcore_barrier()

            pltpu.sync_copy(st_cnt, cntv)
            qref[0] = jnp.int32(0)

            def consume(t, _t):
                cnt_t = cntv[pl.ds(t * _L, _L)][0]
                for ch in range(_SCH // 512):
                    @pl.when(ch * 512 < cnt_t)
                    def _():
                        co = ch * 512
                        to = pl.multiple_of(t * _CCAP + co, 8)
                        pltpu.sync_copy(st_src.at[pl.ds(to, 512)],
                                        sbuf.at[pl.ds(co, 512)])
                        pltpu.sync_copy(st_ldst.at[pl.ds(to, 512)],
                                        dbuf.at[pl.ds(co, 512)])
                        pltpu.sync_copy(st_e0.at[pl.ds(to, 512)],
                                        e0buf.at[pl.ds(co, 512)])
                        pltpu.sync_copy(st_e1.at[pl.ds(to, 512)],
                                        e1buf.at[pl.ds(co, 512)])

                def filt(i, off):
                    ld = dbuf[pl.ds(i * _L, _L)]
                    m = (ld >= olo) & (ld < olo + _R)
                    plsc.store_compressed(q_src.at[pl.ds(off, _L)],
                                          sbuf[pl.ds(i * _L, _L)], mask=m)
                    plsc.store_compressed(q_ldst.at[pl.ds(off, _L)],
                                          ld - olo, mask=m)
                    plsc.store_compressed(q_e0.at[pl.ds(off, _L)],
                                          e0buf[pl.ds(i * _L, _L)], mask=m)
                    plsc.store_compressed(q_e1.at[pl.ds(off, _L)],
                                          e1buf[pl.ds(i * _L, _L)], mask=m)
                    return off + jnp.sum(m.astype(jnp.int32))

                for blk in range(_SCH // _L // 8):
                    @pl.when(blk * 8 * _L < cnt_t)
                    def _():
                        off = lax.fori_loop(blk * 8, blk * 8 + 8,
                                            filt, qref[0])
                        qref[0] = off
                qref[0] = flush(qref[0])
                return 0

            lax.fori_loop(0, _NS, consume, 0)
            plsc.subcore_barrier()
            return 0

        lax.fori_loop(0, _SS, do_ss, 0)
        base_row = pl.multiple_of((2 * p + cid) * _CHC + sid * _R, 8)
        pltpu.sync_copy(acc.at[pl.ds(0, _R)], num_hbm.at[pl.ds(base_row, _R)])
        pltpu.sync_copy(accd.at[pl.ds(0, _R)],
                        den_hbm.at[pl.ds(base_row, _R)])
        return 0

    lax.fori_loop(0, _NPASS, do_pass, 0)


def _edge_msg(h_src, src, dst, ex):
    return pl.kernel(
        _msg_body,
        out_type=[
            jax.ShapeDtypeStruct((_NPAD, _D), jnp.float32),
            jax.ShapeDtypeStruct((_NPAD, _L), jnp.float32),
        ],
        mesh=plsc.VectorSubcoreMesh(core_axis_name="c", subcore_axis_name="s"),
        compiler_params=pltpu.CompilerParams(needs_layout_passes=False),
        scratch_types=[
            pltpu.VMEM((_SCH,), jnp.int32),
            pltpu.VMEM((_SCH,), jnp.int32),
            pltpu.VMEM((_SCH,), jnp.float32),
            pltpu.VMEM((_SCH,), jnp.float32),
            pltpu.VMEM((_CCAP,), jnp.int32),
            pltpu.VMEM((_CCAP,), jnp.int32),
            pltpu.VMEM((_CCAP,), jnp.float32),
            pltpu.VMEM((_CCAP,), jnp.float32),
            pltpu.VMEM((_QCAP,), jnp.int32),
            pltpu.VMEM((_QCAP,), jnp.int32),
            pltpu.VMEM((_QCAP,), jnp.float32),
            pltpu.VMEM((_QCAP,), jnp.float32),
            pltpu.VMEM((_KB, _D), jnp.float32),
            pltpu.VMEM((_L,), jnp.int32),
            pltpu.VMEM((_NS * _L,), jnp.int32),
            pltpu.VMEM((_R + 8, _D), jnp.float32),
            pltpu.VMEM((_R + 8, _L), jnp.float32),
            pltpu.SMEM((1,), jnp.int32),
            pltpu.VMEM_SHARED((_NS * _CCAP,), jnp.int32),
            pltpu.VMEM_SHARED((_NS * _CCAP,), jnp.int32),
            pltpu.VMEM_SHARED((_NS * _CCAP,), jnp.float32),
            pltpu.VMEM_SHARED((_NS * _CCAP,), jnp.float32),
            pltpu.VMEM_SHARED((_NS * _L,), jnp.int32),
            pltpu.SemaphoreType.DMA,
        ],
    )(h_src, src, dst, ex)


def _gat(x_src, x_dst, edge_index, W, att_src, att_dst, bias, num_dst):
    src, dst = edge_index[0], edge_index[1]
    h_src, a_s = _project(x_src, W, att_src, att_dst)
    _, a_d = _project(x_dst, W, att_src, att_dst)
    a_src_t = jnp.pad(a_s[:, :H].T, ((0, 0), (0, _NT - a_s.shape[0])))
    a_dst_t = jnp.pad(a_d[:, H:].T, ((0, 0), (0, _NT - a_d.shape[0])))
    b = a_src_t.max(axis=1) + a_dst_t.max(axis=1)
    b = jnp.where(b > 0, b, 0.2 * b)
    bmat = jnp.broadcast_to(b[:, None], (H, _L)).astype(jnp.float32)
    ex = _edge_ex(a_src_t, a_dst_t, src, dst, bmat)
    if False:
        ex_e = ex.T
        den = jax.ops.segment_sum(ex_e, dst, num_segments=num_dst)
        msg = h_src.reshape(-1, H, C)[src] * ex_e[:, :, None]
        out = jax.ops.segment_sum(msg, dst, num_segments=num_dst)
        num = out.transpose(0, 2, 1).reshape(num_dst, H * C)
        num = jnp.concatenate([out[:, 0, :], out[:, 1, :]], axis=-1)
        den = jnp.pad(den, ((0, 0), (0, _L - H)))
        return num, den
    num, den = _edge_msg(h_src, src, dst, ex)
    return num[:num_dst], den[:num_dst]


def _epilogue_body(n_ref, d_ref, bias_ref, w_ref, b_ref, o_ref):
    n = n_ref[...]
    d = d_ref[...]
    d0 = d[:, 0:1]
    d1 = d[:, 1:2]
    x = ((n[:, :C] / (d0 + 1e-16) + n[:, C:] / (d1 + 1e-16)) * 0.5
         + bias_ref[...])
    mu = x.mean(axis=-1, keepdims=True)
    var = ((x - mu) ** 2).mean(axis=-1, keepdims=True)
    y = (x - mu) * jax.lax.rsqrt(var + EPS) * w_ref[...] + b_ref[...]
    o_ref[...] = jnp.maximum(y, 0.0)


def _epilogue(num, den, bias, w, b):
    n = num.shape[0]
    return pl.pallas_call(
        _epilogue_body,
        grid=(n // _BM,),
        in_specs=[
            pl.BlockSpec((_BM, _D), lambda i: (i, 0)),
            pl.BlockSpec((_BM, _L), lambda i: (i, 0)),
            pl.BlockSpec((1, C), lambda i: (0, 0)),
            pl.BlockSpec((1, C), lambda i: (0, 0)),
            pl.BlockSpec((1, C), lambda i: (0, 0)),
        ],
        out_specs=pl.BlockSpec((_BM, C), lambda i: (i, 0)),
        out_shape=jax.ShapeDtypeStruct((n, C), jnp.float32),
    )(num, den, bias.reshape(1, C), w.reshape(1, C), b.reshape(1, C))


def kernel(x_user, x_item, edge_index_ui, edge_index_iu, W_ui, att_src_ui,
           att_dst_ui, bias_ui, W_iu, att_src_iu, att_dst_iu, bias_iu,
           ln_w_user, ln_b_user, ln_w_item, ln_b_item):
    i_num, i_den = _gat(x_user, x_item, edge_index_ui, W_ui, att_src_ui,
                        att_dst_ui, bias_ui, x_item.shape[0])
    u_num, u_den = _gat(x_item, x_user, edge_index_iu, W_iu, att_src_iu,
                        att_dst_iu, bias_iu, x_user.shape[0])
    user_out = _epilogue(u_num, u_den, bias_iu, ln_w_user, ln_b_user)
    item_out = _epilogue(i_num, i_den, bias_ui, ln_w_item, ln_b_item)
    return (user_out, item_out)


# speedup vs baseline: 14.9586x; 14.9586x over previous
import jax
import jax.numpy as jnp
from jax import lax
from jax.experimental import pallas as pl
from jax.experimental.pallas import tpu as pltpu
from jax.experimental.pallas import tpu_sc as plsc

N_NODE = 50000
C = 128
H = 2
EPS = 1e-5

_BM = 1000

_NC = 2
_NS = 16
_L = 16
_NW = _NC * _NS
_E = 524288
_EPT = _E // _NW
_ECH = 4096
_NT = 50048


def _ex_body(asrc_hbm, adst_hbm, src_hbm, dst_hbm, b_hbm, ex_hbm,
             tsrc, tdst, sbuf, dbuf, exbuf, bbuf):
    cid = lax.axis_index("c")
    sid = lax.axis_index("s")
    wid = sid * _NC + cid
    base = wid * _EPT
    pltpu.sync_copy(b_hbm, bbuf)
    for h in range(H):
        pltpu.sync_copy(asrc_hbm.at[h], tsrc)
        pltpu.sync_copy(adst_hbm.at[h], tdst)
        bvec = bbuf[h]
        for ch in range(_EPT // _ECH):
            off = base + ch * _ECH
            pltpu.sync_copy(src_hbm.at[pl.ds(off, _ECH)], sbuf)
            pltpu.sync_copy(dst_hbm.at[pl.ds(off, _ECH)], dbuf)

            def body(i, _):
                s = sbuf[pl.ds(i * _L, _L)]
                d = dbuf[pl.ds(i * _L, _L)]
                al = plsc.load_gather(tsrc, [s]) + plsc.load_gather(tdst, [d])
                al = jnp.where(al > 0, al, 0.2 * al) - bvec
                exbuf[pl.ds(i * _L, _L)] = jnp.exp(al)
                return 0

            lax.fori_loop(0, _ECH // _L, body, 0)
            pltpu.sync_copy(exbuf, ex_hbm.at[h, pl.ds(off, _ECH)])


def _edge_ex(a_src_t, a_dst_t, src, dst, bmat):
    return pl.kernel(
        _ex_body,
        out_type=jax.ShapeDtypeStruct((H, _E), jnp.float32),
        mesh=plsc.VectorSubcoreMesh(core_axis_name="c", subcore_axis_name="s"),
        compiler_params=pltpu.CompilerParams(needs_layout_passes=False),
        scratch_types=[
            pltpu.VMEM((_NT,), jnp.float32),
            pltpu.VMEM((_NT,), jnp.float32),
            pltpu.VMEM((_ECH,), jnp.int32),
            pltpu.VMEM((_ECH,), jnp.int32),
            pltpu.VMEM((_ECH,), jnp.float32),
            pltpu.VMEM((H, _L), jnp.float32),
        ],
    )(a_src_t, a_dst_t, src, dst, bmat)


def _proj_body(x_ref, w_ref, att_src_ref, att_dst_ref, h_ref, a_ref):
    h = jnp.dot(x_ref[...], w_ref[...], preferred_element_type=jnp.float32)
    h_ref[...] = h
    hh = h.reshape(-1, H, C)
    a = (hh * att_dst_ref[...]).sum(-1)
    b = (hh * att_src_ref[...]).sum(-1)
    a_ref[...] = jnp.concatenate([b, a], axis=-1)


def _project(x, W, att_src, att_dst):
    n = x.shape[0]
    h, a = pl.pallas_call(
        _proj_body,
        grid=(n // _BM,),
        in_specs=[
            pl.BlockSpec((_BM, C), lambda i: (i, 0)),
            pl.BlockSpec((C, H * C), lambda i: (0, 0)),
            pl.BlockSpec((1, H, C), lambda i: (0, 0, 0)),
            pl.BlockSpec((1, H, C), lambda i: (0, 0, 0)),
        ],
        out_specs=[
            pl.BlockSpec((_BM, H * C), lambda i: (i, 0)),
            pl.BlockSpec((_BM, 2 * H), lambda i: (i, 0)),
        ],
        out_shape=[
            jax.ShapeDtypeStruct((n, H * C), jnp.float32),
            jax.ShapeDtypeStruct((n, 2 * H), jnp.float32),
        ],
    )(x, W, att_src, att_dst)
    return h, a


_R = 168
_CHC = _NS * _R
_NPASS = 10
_NPAD = 2 * _NPASS * _CHC
_SCH = 2048
_SS = (_E // _NS) // _SCH
_CCAP = _SCH + _L
_QT = 1024
_QCAP = _QT + _SCH + _L
_KB = 64
_D = H * C


def _msg_body(hsrc_hbm, src_hbm, dst_hbm, ex_hbm, num_hbm, den_hbm,
              dbuf, sbuf, e0buf, e1buf, c_src, c_ldst, c_e0, c_e1,
              q_src, q_ldst, q_e0, q_e1, g, cbuf, cntv, acc, accd, qref,
              st_src, st_ldst, st_e0, st_e1, st_cnt, sem):
    cid = lax.axis_index("c")
    sid = lax.axis_index("s")
    lane = lax.iota(jnp.int32, _L)
    zi = jnp.zeros((_L,), jnp.int32)
    zf = jnp.zeros((_L,), jnp.float32)
    qfill = jnp.full((_L,), _R, jnp.int32)

    def prefill_q(i, _):
        q_src[pl.ds(i * _L, _L)] = zi
        q_ldst[pl.ds(i * _L, _L)] = qfill
        return 0

    def flush(qoff):
        def batch(b, _):
            @pl.when(b * _KB < qoff)
            def _():
                pltpu.async_copy(
                    hsrc_hbm.at[q_src.at[pl.ds(b * _KB, _KB)]], g, sem
                ).wait()

                def accum(j, _):
                    jj = b * _KB + j
                    gidx = jnp.full((_L,), jj, jnp.int32)
                    e0 = plsc.load_gather(q_e0, [gidx])
                    e1 = plsc.load_gather(q_e1, [gidx])
                    r = plsc.load_gather(q_ldst, [gidx])[0]
                    for q in range(8):
                        plsc.addupdate(acc.at[r, pl.ds(q * _L, _L)],
                                       g[j, pl.ds(q * _L, _L)] * e0)
                    for q in range(8, 16):
                        plsc.addupdate(acc.at[r, pl.ds(q * _L, _L)],
                                       g[j, pl.ds(q * _L, _L)] * e1)
                    dv = (jnp.where(lane == 0, e0, 0.0)
                          + jnp.where(lane == 1, e1, 0.0))
                    plsc.addupdate(accd.at[r], dv)
                    return 0

                lax.fori_loop(0, _KB, accum, 0)
            return 0

        lax.fori_loop(0, _QCAP // _KB + 1, batch, 0)
        lax.fori_loop(0, _QCAP // _L, prefill_q, 0)
        return jnp.int32(0)

    def do_pass(p, _p):
        klo = (2 * p + cid) * _CHC
        olo = sid * _R

        def zacc(r, _):
            for q in range(_D // _L):
                acc[r, pl.ds(q * _L, _L)] = zf
            accd[r, :] = zf
            return 0

        lax.fori_loop(0, _R + 8, zacc, 0)
        lax.fori_loop(0, _QCAP // _L, prefill_q, 0)

        def do_ss(ss, _s):
            ebase = pl.multiple_of(sid * (_E // _NS) + ss * _SCH, 8)
            pltpu.sync_copy(dst_hbm.at[pl.ds(ebase, _SCH)], dbuf)
            pltpu.sync_copy(src_hbm.at[pl.ds(ebase, _SCH)], sbuf)
            pltpu.sync_copy(ex_hbm.at[0, pl.ds(ebase, _SCH)], e0buf)
            pltpu.sync_copy(ex_hbm.at[1, pl.ds(ebase, _SCH)], e1buf)

            def prefill_c(i, _):
                c_src[pl.ds(i * _L, _L)] = zi
                c_ldst[pl.ds(i * _L, _L)] = jnp.full((_L,), _CHC, jnp.int32)
                return 0

            lax.fori_loop(0, _CCAP // _L, prefill_c, 0)

            def scan(i, off):
                d = dbuf[pl.ds(i * _L, _L)]
                m = (d >= klo) & (d < klo + _CHC)
                plsc.store_compressed(c_src.at[pl.ds(off, _L)],
                                      sbuf[pl.ds(i * _L, _L)], mask=m)
                plsc.store_compressed(c_ldst.at[pl.ds(off, _L)],
                                      d - klo, mask=m)
                plsc.store_compressed(c_e0.at[pl.ds(off, _L)],
                                      e0buf[pl.ds(i * _L, _L)], mask=m)
                plsc.store_compressed(c_e1.at[pl.ds(off, _L)],
                                      e1buf[pl.ds(i * _L, _L)], mask=m)
                return off + jnp.sum(m.astype(jnp.int32))

            cnt = lax.fori_loop(0, _SCH // _L, scan, jnp.int32(0))
            so = pl.multiple_of(sid * _CCAP, 8)
            pltpu.sync_copy(c_src, st_src.at[pl.ds(so, _CCAP)])
            pltpu.sync_copy(c_ldst, st_ldst.at[pl.ds(so, _CCAP)])
            pltpu.sync_copy(c_e0, st_e0.at[pl.ds(so, _CCAP)])
            pltpu.sync_copy(c_e1, st_e1.at[pl.ds(so, _CCAP)])
            cbuf[...] = jnp.full((_L,), cnt, jnp.int32)
            pltpu.sync_copy(cbuf, st_cnt.at[pl.ds(sid * _L, _L)])
            plsc.subcore_barrier()

            pltpu.sync_copy(st_cnt, cntv)
            qref[0] = jnp.int32(0)

            def consume(t, _t):
                cnt_t = cntv[pl.ds(t * _L, _L)][0]
                for ch in range(_SCH // 512):
                    @pl.when(ch * 512 < cnt_t)
                    def _():
                        co = ch * 512
                        to = pl.multiple_of(t * _CCAP + co, 8)
                        pltpu.sync_copy(st_src.at[pl.ds(to, 512)],
                                        sbuf.at[pl.ds(co, 512)])
                        pltpu.sync_copy(st_ldst.at[pl.ds(to, 512)],
                                        dbuf.at[pl.ds(co, 512)])
                        pltpu.sync_copy(st_e0.at[pl.ds(to, 512)],
                                        e0buf.at[pl.ds(co, 512)])
                        pltpu.sync_copy(st_e1.at[pl.ds(to, 512)],
                                        e1buf.at[pl.ds(co, 512)])

                def filt(i, off):
                    ld = dbuf[pl.ds(i * _L, _L)]
                    m = (ld >= olo) & (ld < olo + _R)
                    plsc.store_compressed(q_src.at[pl.ds(off, _L)],
                                          sbuf[pl.ds(i * _L, _L)], mask=m)
                    plsc.store_compressed(q_ldst.at[pl.ds(off, _L)],
                                          ld - olo, mask=m)
                    plsc.store_compressed(q_e0.at[pl.ds(off, _L)],
                                          e0buf[pl.ds(i * _L, _L)], mask=m)
                    plsc.store_compressed(q_e1.at[pl.ds(off, _L)],
                                          e1buf[pl.ds(i * _L, _L)], mask=m)
                    return off + jnp.sum(m.astype(jnp.int32))

                for blk in range(_SCH // _L // 8):
                    @pl.when(blk * 8 * _L < cnt_t)
                    def _():
                        off = lax.fori_loop(blk * 8, blk * 8 + 8,
                                            filt, qref[0])
                        qref[0] = off
                return 0

            def consume_guarded(t, _t):
                @pl.when(qref[0] >= _QT)
                def _():
                    qref[0] = flush(qref[0])
                return consume(t, _t)

            lax.fori_loop(0, _NS, consume_guarded, 0)
            qref[0] = flush(qref[0])
            plsc.subcore_barrier()
            return 0

        lax.fori_loop(0, _SS, do_ss, 0)
        base_row = pl.multiple_of((2 * p + cid) * _CHC + sid * _R, 8)
        pltpu.sync_copy(acc.at[pl.ds(0, _R)], num_hbm.at[pl.ds(base_row, _R)])
        pltpu.sync_copy(accd.at[pl.ds(0, _R)],
                        den_hbm.at[pl.ds(base_row, _R)])
        return 0

    lax.fori_loop(0, _NPASS, do_pass, 0)


def _edge_msg(h_src, src, dst, ex):
    return pl.kernel(
        _msg_body,
        out_type=[
            jax.ShapeDtypeStruct((_NPAD, _D), jnp.float32),
            jax.ShapeDtypeStruct((_NPAD, _L), jnp.float32),
        ],
        mesh=plsc.VectorSubcoreMesh(core_axis_name="c", subcore_axis_name="s"),
        compiler_params=pltpu.CompilerParams(needs_layout_passes=False),
        scratch_types=[
            pltpu.VMEM((_SCH,), jnp.int32),
            pltpu.VMEM((_SCH,), jnp.int32),
            pltpu.VMEM((_SCH,), jnp.float32),
            pltpu.VMEM((_SCH,), jnp.float32),
            pltpu.VMEM((_CCAP,), jnp.int32),
            pltpu.VMEM((_CCAP,), jnp.int32),
            pltpu.VMEM((_CCAP,), jnp.float32),
            pltpu.VMEM((_CCAP,), jnp.float32),
            pltpu.VMEM((_QCAP,), jnp.int32),
            pltpu.VMEM((_QCAP,), jnp.int32),
            pltpu.VMEM((_QCAP,), jnp.float32),
            pltpu.VMEM((_QCAP,), jnp.float32),
            pltpu.VMEM((_KB, _D), jnp.float32),
            pltpu.VMEM((_L,), jnp.int32),
            pltpu.VMEM((_NS * _L,), jnp.int32),
            pltpu.VMEM((_R + 8, _D), jnp.float32),
            pltpu.VMEM((_R + 8, _L), jnp.float32),
            pltpu.SMEM((1,), jnp.int32),
            pltpu.VMEM_SHARED((_NS * _CCAP,), jnp.int32),
            pltpu.VMEM_SHARED((_NS * _CCAP,), jnp.int32),
            pltpu.VMEM_SHARED((_NS * _CCAP,), jnp.float32),
            pltpu.VMEM_SHARED((_NS * _CCAP,), jnp.float32),
            pltpu.VMEM_SHARED((_NS * _L,), jnp.int32),
            pltpu.SemaphoreType.DMA,
        ],
    )(h_src, src, dst, ex)


def _gat(x_src, x_dst, edge_index, W, att_src, att_dst, bias, num_dst):
    src, dst = edge_index[0], edge_index[1]
    h_src, a_s = _project(x_src, W, att_src, att_dst)
    _, a_d = _project(x_dst, W, att_src, att_dst)
    a_src_t = jnp.pad(a_s[:, :H].T, ((0, 0), (0, _NT - a_s.shape[0])))
    a_dst_t = jnp.pad(a_d[:, H:].T, ((0, 0), (0, _NT - a_d.shape[0])))
    b = a_src_t.max(axis=1) + a_dst_t.max(axis=1)
    b = jnp.where(b > 0, b, 0.2 * b)
    bmat = jnp.broadcast_to(b[:, None], (H, _L)).astype(jnp.float32)
    ex = _edge_ex(a_src_t, a_dst_t, src, dst, bmat)
    if False:
        ex_e = ex.T
        den = jax.ops.segment_sum(ex_e, dst, num_segments=num_dst)
        msg = h_src.reshape(-1, H, C)[src] * ex_e[:, :, None]
        out = jax.ops.segment_sum(msg, dst, num_segments=num_dst)
        num = out.transpose(0, 2, 1).reshape(num_dst, H * C)
        num = jnp.concatenate([out[:, 0, :], out[:, 1, :]], axis=-1)
        den = jnp.pad(den, ((0, 0), (0, _L - H)))
        return num, den
    num, den = _edge_msg(h_src, src, dst, ex)
    return num[:num_dst], den[:num_dst]


def _epilogue_body(n_ref, d_ref, bias_ref, w_ref, b_ref, o_ref):
    n = n_ref[...]
    d = d_ref[...]
    d0 = d[:, 0:1]
    d1 = d[:, 1:2]
    x = ((n[:, :C] / (d0 + 1e-16) + n[:, C:] / (d1 + 1e-16)) * 0.5
         + bias_ref[...])
    mu = x.mean(axis=-1, keepdims=True)
    var = ((x - mu) ** 2).mean(axis=-1, keepdims=True)
    y = (x - mu) * jax.lax.rsqrt(var + EPS) * w_ref[...] + b_ref[...]
    o_ref[...] = jnp.maximum(y, 0.0)


def _epilogue(num, den, bias, w, b):
    n = num.shape[0]
    return pl.pallas_call(
        _epilogue_body,
        grid=(n // _BM,),
        in_specs=[
            pl.BlockSpec((_BM, _D), lambda i: (i, 0)),
            pl.BlockSpec((_BM, _L), lambda i: (i, 0)),
            pl.BlockSpec((1, C), lambda i: (0, 0)),
            pl.BlockSpec((1, C), lambda i: (0, 0)),
            pl.BlockSpec((1, C), lambda i: (0, 0)),
        ],
        out_specs=pl.BlockSpec((_BM, C), lambda i: (i, 0)),
        out_shape=jax.ShapeDtypeStruct((n, C), jnp.float32),
    )(num, den, bias.reshape(1, C), w.reshape(1, C), b.reshape(1, C))


def kernel(x_user, x_item, edge_index_ui, edge_index_iu, W_ui, att_src_ui,
           att_dst_ui, bias_ui, W_iu, att_src_iu, att_dst_iu, bias_iu,
           ln_w_user, ln_b_user, ln_w_item, ln_b_item):
    i_num, i_den = _gat(x_user, x_item, edge_index_ui, W_ui, att_src_ui,
                        att_dst_ui, bias_ui, x_item.shape[0])
    u_num, u_den = _gat(x_item, x_user, edge_index_iu, W_iu, att_src_iu,
                        att_dst_iu, bias_iu, x_user.shape[0])
    user_out = _epilogue(u_num, u_den, bias_iu, ln_w_user, ln_b_user)
    item_out = _epilogue(i_num, i_den, bias_ui, ln_w_item, ln_b_item)
    return (user_out, item_out)


# speedup vs baseline: 14.9756x; 1.0011x over previous
import jax
import jax.numpy as jnp
from jax import lax
from jax.experimental import pallas as pl
from jax.experimental.pallas import tpu as pltpu
from jax.experimental.pallas import tpu_sc as plsc

N_NODE = 50000
C = 128
H = 2
EPS = 1e-5

_BM = 1000

_NC = 2
_NS = 16
_L = 16
_NW = _NC * _NS
_E = 524288
_EPT = _E // _NW
_ECH = 4096
_NT = 50048


def _ex_body(asrc_hbm, adst_hbm, src_hbm, dst_hbm, b_hbm, ex_hbm,
             tsrc, tdst, sbuf, dbuf, exbuf, bbuf):
    cid = lax.axis_index("c")
    sid = lax.axis_index("s")
    wid = sid * _NC + cid
    base = wid * _EPT
    pltpu.sync_copy(b_hbm, bbuf)
    for h in range(H):
        pltpu.sync_copy(asrc_hbm.at[h], tsrc)
        pltpu.sync_copy(adst_hbm.at[h], tdst)
        bvec = bbuf[h]
        for ch in range(_EPT // _ECH):
            off = base + ch * _ECH
            pltpu.sync_copy(src_hbm.at[pl.ds(off, _ECH)], sbuf)
            pltpu.sync_copy(dst_hbm.at[pl.ds(off, _ECH)], dbuf)

            def body(i, _):
                s = sbuf[pl.ds(i * _L, _L)]
                d = dbuf[pl.ds(i * _L, _L)]
                al = plsc.load_gather(tsrc, [s]) + plsc.load_gather(tdst, [d])
                al = jnp.where(al > 0, al, 0.2 * al) - bvec
                exbuf[pl.ds(i * _L, _L)] = jnp.exp(al)
                return 0

            lax.fori_loop(0, _ECH // _L, body, 0)
            pltpu.sync_copy(exbuf, ex_hbm.at[h, pl.ds(off, _ECH)])


def _edge_ex(a_src_t, a_dst_t, src, dst, bmat):
    return pl.kernel(
        _ex_body,
        out_type=jax.ShapeDtypeStruct((H, _E), jnp.float32),
        mesh=plsc.VectorSubcoreMesh(core_axis_name="c", subcore_axis_name="s"),
        compiler_params=pltpu.CompilerParams(needs_layout_passes=False),
        scratch_types=[
            pltpu.VMEM((_NT,), jnp.float32),
            pltpu.VMEM((_NT,), jnp.float32),
            pltpu.VMEM((_ECH,), jnp.int32),
            pltpu.VMEM((_ECH,), jnp.int32),
            pltpu.VMEM((_ECH,), jnp.float32),
            pltpu.VMEM((H, _L), jnp.float32),
        ],
    )(a_src_t, a_dst_t, src, dst, bmat)


def _proj_body(x_ref, w_ref, att_src_ref, att_dst_ref, h_ref, a_ref):
    h = jnp.dot(x_ref[...], w_ref[...], preferred_element_type=jnp.float32)
    h_ref[...] = h
    hh = h.reshape(-1, H, C)
    a = (hh * att_dst_ref[...]).sum(-1)
    b = (hh * att_src_ref[...]).sum(-1)
    a_ref[...] = jnp.concatenate([b, a], axis=-1)


def _project(x, W, att_src, att_dst):
    n = x.shape[0]
    h, a = pl.pallas_call(
        _proj_body,
        grid=(n // _BM,),
        in_specs=[
            pl.BlockSpec((_BM, C), lambda i: (i, 0)),
            pl.BlockSpec((C, H * C), lambda i: (0, 0)),
            pl.BlockSpec((1, H, C), lambda i: (0, 0, 0)),
            pl.BlockSpec((1, H, C), lambda i: (0, 0, 0)),
        ],
        out_specs=[
            pl.BlockSpec((_BM, H * C), lambda i: (i, 0)),
            pl.BlockSpec((_BM, 2 * H), lambda i: (i, 0)),
        ],
        out_shape=[
            jax.ShapeDtypeStruct((n, H * C), jnp.float32),
            jax.ShapeDtypeStruct((n, 2 * H), jnp.float32),
        ],
    )(x, W, att_src, att_dst)
    return h, a


_R = 168
_CHC = _NS * _R
_NPASS = 10
_NPAD = 2 * _NPASS * _CHC
_SCH = 2048
_SS = (_E // _NS) // _SCH
_CCAP = _SCH + _L
_QT = 1024
_QCAP = _QT + _SCH + _L
_KB = 64
_D = H * C


def _msg_body(hsrc_hbm, src_hbm, dst_hbm, ex_hbm, num_hbm, den_hbm,
              dbuf, sbuf, e0buf, e1buf, c_src, c_ldst, c_e0, c_e1,
              q_src, q_ldst, q_e0, q_e1, g, cbuf, cntv, acc, accd, qref,
              st_src, st_ldst, st_e0, st_e1, st_cnt, sem):
    cid = lax.axis_index("c")
    sid = lax.axis_index("s")
    lane = lax.iota(jnp.int32, _L)
    zi = jnp.zeros((_L,), jnp.int32)
    zf = jnp.zeros((_L,), jnp.float32)
    qfill = jnp.full((_L,), _R, jnp.int32)

    def prefill_q(i, _):
        q_src[pl.ds(i * _L, _L)] = zi
        q_ldst[pl.ds(i * _L, _L)] = qfill
        return 0

    def flush(qoff):
        def batch(b, _):
            @pl.when(b * _KB < qoff)
            def _():
                pltpu.async_copy(
                    hsrc_hbm.at[q_src.at[pl.ds(b * _KB, _KB)]], g, sem
                ).wait()

                def accum(j, _):
                    jj = b * _KB + j
                    gidx = jnp.full((_L,), jj, jnp.int32)
                    e0 = plsc.load_gather(q_e0, [gidx])
                    e1 = plsc.load_gather(q_e1, [gidx])
                    r = plsc.load_gather(q_ldst, [gidx])[0]
                    for q in range(8):
                        plsc.addupdate(acc.at[r, pl.ds(q * _L, _L)],
                                       g[j, pl.ds(q * _L, _L)] * e0)
                    for q in range(8, 16):
                        plsc.addupdate(acc.at[r, pl.ds(q * _L, _L)],
                                       g[j, pl.ds(q * _L, _L)] * e1)
                    dv = (jnp.where(lane == 0, e0, 0.0)
                          + jnp.where(lane == 1, e1, 0.0))
                    plsc.addupdate(accd.at[r], dv)
                    return 0

                lax.fori_loop(0, _KB, accum, 0)
            return 0

        lax.fori_loop(0, _QCAP // _KB + 1, batch, 0)
        lax.fori_loop(0, _QCAP // _L, prefill_q, 0)
        return jnp.int32(0)

    def do_pass(p, _p):
        klo = (2 * p + cid) * _CHC
        olo = sid * _R

        def zacc(r, _):
            for q in range(_D // _L):
                acc[r, pl.ds(q * _L, _L)] = zf
            accd[r, :] = zf
            return 0

        lax.fori_loop(0, _R + 8, zacc, 0)
        lax.fori_loop(0, _QCAP // _L, prefill_q, 0)

        def do_ss(ss, _s):
            ebase = pl.multiple_of(sid * (_E // _NS) + ss * _SCH, 8)
            pltpu.sync_copy(dst_hbm.at[pl.ds(ebase, _SCH)], dbuf)
            pltpu.sync_copy(src_hbm.at[pl.ds(ebase, _SCH)], sbuf)
            pltpu.sync_copy(ex_hbm.at[0, pl.ds(ebase, _SCH)], e0buf)
            pltpu.sync_copy(ex_hbm.at[1, pl.ds(ebase, _SCH)], e1buf)

            def prefill_c(i, _):
                c_src[pl.ds(i * _L, _L)] = zi
                c_ldst[pl.ds(i * _L, _L)] = jnp.full((_L,), _CHC, jnp.int32)
                return 0

            lax.fori_loop(0, _CCAP // _L, prefill_c, 0)

            def scan(i, off):
                d = dbuf[pl.ds(i * _L, _L)]
                m = (d >= klo) & (d < klo + _CHC)
                plsc.store_compressed(c_src.at[pl.ds(off, _L)],
                                      sbuf[pl.ds(i * _L, _L)], mask=m)
                plsc.store_compressed(c_ldst.at[pl.ds(off, _L)],
                                      d - klo, mask=m)
                plsc.store_compressed(c_e0.at[pl.ds(off, _L)],
                                      e0buf[pl.ds(i * _L, _L)], mask=m)
                plsc.store_compressed(c_e1.at[pl.ds(off, _L)],
                                      e1buf[pl.ds(i * _L, _L)], mask=m)
                return off + jnp.sum(m.astype(jnp.int32))

            cnt = lax.fori_loop(0, _SCH // _L, scan, jnp.int32(0))
            so = pl.multiple_of(sid * _CCAP, 8)
            pltpu.sync_copy(c_src, st_src.at[pl.ds(so, _CCAP)])
            pltpu.sync_copy(c_ldst, st_ldst.at[pl.ds(so, _CCAP)])
            pltpu.sync_copy(c_e0, st_e0.at[pl.ds(so, _CCAP)])
            pltpu.sync_copy(c_e1, st_e1.at[pl.ds(so, _CCAP)])
            cbuf[...] = jnp.full((_L,), cnt, jnp.int32)
            pltpu.sync_copy(cbuf, st_cnt.at[pl.ds(sid * _L, _L)])
            plsc.subcore_barrier()

            pltpu.sync_copy(st_cnt, cntv)
            qref[0] = jnp.int32(0)

            def consume(t, _t):
                cnt_t = cntv[pl.ds(t * _L, _L)][0]
                for ch in range(_SCH // 512):
                    @pl.when(ch * 512 < cnt_t)
                    def _():
                        co = ch * 512
                        to = pl.multiple_of(t * _CCAP + co, 8)
                        pltpu.sync_copy(st_src.at[pl.ds(to, 512)],
                                        sbuf.at[pl.ds(co, 512)])
                        pltpu.sync_copy(st_ldst.at[pl.ds(to, 512)],
                                        dbuf.at[pl.ds(co, 512)])
                        pltpu.sync_copy(st_e0.at[pl.ds(to, 512)],
                                        e0buf.at[pl.ds(co, 512)])
                        pltpu.sync_copy(st_e1.at[pl.ds(to, 512)],
                                        e1buf.at[pl.ds(co, 512)])

                def filt(i, off):
                    ld = dbuf[pl.ds(i * _L, _L)]
                    m = (ld >= olo) & (ld < olo + _R)
                    plsc.store_compressed(q_src.at[pl.ds(off, _L)],
                                          sbuf[pl.ds(i * _L, _L)], mask=m)
                    plsc.store_compressed(q_ldst.at[pl.ds(off, _L)],
                                          ld - olo, mask=m)
                    plsc.store_compressed(q_e0.at[pl.ds(off, _L)],
                                          e0buf[pl.ds(i * _L, _L)], mask=m)
                    plsc.store_compressed(q_e1.at[pl.ds(off, _L)],
                                          e1buf[pl.ds(i * _L, _L)], mask=m)
                    return off + jnp.sum(m.astype(jnp.int32))

                for blk in range(_SCH // _L // 8):
                    @pl.when(blk * 8 * _L < cnt_t)
                    def _():
                        off = lax.fori_loop(blk * 8, blk * 8 + 8,
                                            filt, qref[0])
                        qref[0] = off
                return 0

            def consume_guarded(t, _t):
                @pl.when(qref[0] >= _QT)
                def _():
                    qref[0] = flush(qref[0])
                return consume(t, _t)

            lax.fori_loop(0, _NS, consume_guarded, 0)
            qref[0] = flush(qref[0])
            plsc.subcore_barrier()
            return 0

        lax.fori_loop(0, _SS, do_ss, 0)
        base_row = pl.multiple_of((2 * p + cid) * _CHC + sid * _R, 8)
        pltpu.sync_copy(acc.at[pl.ds(0, _R)], num_hbm.at[pl.ds(base_row, _R)])
        pltpu.sync_copy(accd.at[pl.ds(0, _R)],
                        den_hbm.at[pl.ds(base_row, _R)])
        return 0

    lax.fori_loop(0, _NPASS, do_pass, 0)


def _edge_msg(h_src, src, dst, ex):
    return pl.kernel(
        _msg_body,
        out_type=[
            jax.ShapeDtypeStruct((_NPAD, _D), jnp.float32),
            jax.ShapeDtypeStruct((_NPAD, _L), jnp.float32),
        ],
        mesh=plsc.VectorSubcoreMesh(core_axis_name="c", subcore_axis_name="s"),
        compiler_params=pltpu.CompilerParams(needs_layout_passes=False),
        scratch_types=[
            pltpu.VMEM((_SCH,), jnp.int32),
            pltpu.VMEM((_SCH,), jnp.int32),
            pltpu.VMEM((_SCH,), jnp.float32),
            pltpu.VMEM((_SCH,), jnp.float32),
            pltpu.VMEM((_CCAP,), jnp.int32),
            pltpu.VMEM((_CCAP,), jnp.int32),
            pltpu.VMEM((_CCAP,), jnp.float32),
            pltpu.VMEM((_CCAP,), jnp.float32),
            pltpu.VMEM((_QCAP,), jnp.int32),
            pltpu.VMEM((_QCAP,), jnp.int32),
            pltpu.VMEM((_QCAP,), jnp.float32),
            pltpu.VMEM((_QCAP,), jnp.float32),
            pltpu.VMEM((_KB, _D), jnp.float32),
            pltpu.VMEM((_L,), jnp.int32),
            pltpu.VMEM((_NS * _L,), jnp.int32),
            pltpu.VMEM((_R + 8, _D), jnp.float32),
            pltpu.VMEM((_R + 8, _L), jnp.float32),
            pltpu.SMEM((1,), jnp.int32),
            pltpu.VMEM_SHARED((_NS * _CCAP,), jnp.int32),
            pltpu.VMEM_SHARED((_NS * _CCAP,), jnp.int32),
            pltpu.VMEM_SHARED((_NS * _CCAP,), jnp.float32),
            pltpu.VMEM_SHARED((_NS * _CCAP,), jnp.float32),
            pltpu.VMEM_SHARED((_NS * _L,), jnp.int32),
            pltpu.SemaphoreType.DMA,
        ],
    )(h_src, src, dst, ex)


def _gat(x_src, x_dst, edge_index, W, att_src, att_dst, bias, num_dst):
    src, dst = edge_index[0], edge_index[1]
    h_src, a_s = _project(x_src, W, att_src, att_dst)
    _, a_d = _project(x_dst, W, att_src, att_dst)
    a_src_t = jnp.pad(a_s[:, :H].T, ((0, 0), (0, _NT - a_s.shape[0])))
    a_dst_t = jnp.pad(a_d[:, H:].T, ((0, 0), (0, _NT - a_d.shape[0])))
    b = a_src_t.max(axis=1) + a_dst_t.max(axis=1)
    b = jnp.where(b > 0, b, 0.2 * b)
    bmat = jnp.broadcast_to(b[:, None], (H, _L)).astype(jnp.float32)
    ex = _edge_ex(a_src_t, a_dst_t, src, dst, bmat)
    num, den = _edge_msg(h_src, src, dst, ex)
    return num[:num_dst], den[:num_dst]


def _epilogue_body(n_ref, d_ref, bias_ref, w_ref, b_ref, o_ref):
    n = n_ref[...]
    d = d_ref[...]
    d0 = d[:, 0:1]
    d1 = d[:, 1:2]
    x = ((n[:, :C] / (d0 + 1e-16) + n[:, C:] / (d1 + 1e-16)) * 0.5
         + bias_ref[...])
    mu = x.mean(axis=-1, keepdims=True)
    var = ((x - mu) ** 2).mean(axis=-1, keepdims=True)
    y = (x - mu) * jax.lax.rsqrt(var + EPS) * w_ref[...] + b_ref[...]
    o_ref[...] = jnp.maximum(y, 0.0)


def _epilogue(num, den, bias, w, b):
    n = num.shape[0]
    return pl.pallas_call(
        _epilogue_body,
        grid=(n // _BM,),
        in_specs=[
            pl.BlockSpec((_BM, _D), lambda i: (i, 0)),
            pl.BlockSpec((_BM, _L), lambda i: (i, 0)),
            pl.BlockSpec((1, C), lambda i: (0, 0)),
            pl.BlockSpec((1, C), lambda i: (0, 0)),
            pl.BlockSpec((1, C), lambda i: (0, 0)),
        ],
        out_specs=pl.BlockSpec((_BM, C), lambda i: (i, 0)),
        out_shape=jax.ShapeDtypeStruct((n, C), jnp.float32),
    )(num, den, bias.reshape(1, C), w.reshape(1, C), b.reshape(1, C))


def kernel(x_user, x_item, edge_index_ui, edge_index_iu, W_ui, att_src_ui,
           att_dst_ui, bias_ui, W_iu, att_src_iu, att_dst_iu, bias_iu,
           ln_w_user, ln_b_user, ln_w_item, ln_b_item):
    i_num, i_den = _gat(x_user, x_item, edge_index_ui, W_ui, att_src_ui,
                        att_dst_ui, bias_ui, x_item.shape[0])
    u_num, u_den = _gat(x_item, x_user, edge_index_iu, W_iu, att_src_iu,
                        att_dst_iu, bias_iu, x_user.shape[0])
    user_out = _epilogue(u_num, u_den, bias_iu, ln_w_user, ln_b_user)
    item_out = _epilogue(i_num, i_den, bias_ui, ln_w_item, ln_b_item)
    return (user_out, item_out)
